# baseline (speedup 1.0000x reference)
import functools

import jax
import jax.numpy as jnp
import numpy as np
from jax import lax
from jax.experimental import pallas as pl
from jax.experimental.pallas import tpu as pltpu
from jax.experimental.pallas import tpu_sc as plsc

F32 = jnp.float32
I32 = jnp.int32

NS_ = [40962, 10242, 2562, 642, 162]
ES_ = [6 * n for n in NS_]
NK = 10
KP = 16
NF_ = [16, 32, 64, 128, 256]
EPS = 1e-15
SPMEM_WORDS = 2097151


def _ru(x, m):
    return (x + m - 1) // m * m


EPAD = [_ru(e, 4096) for e in ES_]
NPAD = [_ru(n + 8, 2048) for n in NS_]
BE_ = [2048, 2048, 1024, 512, 256]
SPLIT = [True, True, True, True, True]

X_NAMES = ['conv1', 'conv1s', 'conv2', 'conv2s', 'conv3', 'conv3s',
           'conv4', 'conv4s', 'conv5', 'conv5s']
Y_NAMES = ['conv1_d', 'conv1s_d', 'conv2_d', 'conv2s_d', 'conv3_d',
           'conv3s_d', 'conv4', 'conv4s', 'conv5', 'conv5s']

_MESH = dict(core_axis_name="c", subcore_axis_name="s")
_CPARAMS = dict(use_tc_tiling_on_sc=False)


def _nbuf(w):
    return max(1, min(8, 376832 // (512 * w)))


@functools.cache
def _gather_call(npad, w2, epad):
    nchunks = epad // 128
    ct = nchunks // 32
    rows_per_tile = npad // 16
    avail = SPMEM_WORDS - npad * w2 - 16 * 1024 - 16384
    nbs = avail // (16 * 128 * w2)
    staged = nbs >= 1
    nb = min(8, nbs) if staged else _nbuf(w2)
    ng = ct // nb
    tail = ct - ng * nb

    scratch = [
        pltpu.VMEM((nb, 128), I32),
        pltpu.VMEM((nb * 128, w2), F32),
        pltpu.SemaphoreType.DMA,
    ]
    if staged:
        scratch.append(pltpu.VMEM_SHARED((npad, w2), F32))

    @functools.partial(
        pl.kernel,
        out_type=jax.ShapeDtypeStruct((epad, w2), F32),
        scratch_types=scratch,
        mesh=plsc.VectorSubcoreMesh(**_MESH),
        compiler_params=pltpu.CompilerParams(**_CPARAMS),
    )
    def gather_k(xb, src2d, gx, idx2, rows_v, sem, *sh):
        cid = lax.axis_index("c")
        sid = lax.axis_index("s")
        if staged:
            sh_x = sh[0]
            base = sid * rows_per_tile
            pltpu.sync_copy(xb.at[pl.ds(base, rows_per_tile)],
                            sh_x.at[pl.ds(base, rows_per_tile)])
            plsc.subcore_barrier()
            src = sh_x
        else:
            src = xb

        def group(r0, k):
            pltpu.sync_copy(src2d.at[pl.ds(r0, k)], idx2.at[pl.ds(0, k)])
            descs = [pltpu.async_copy(src.at[idx2.at[jb]],
                                      rows_v.at[pl.ds(jb * 128, 128)], sem)
                     for jb in range(k)]
            for d in descs:
                d.wait()
            pltpu.sync_copy(rows_v.at[pl.ds(0, k * 128)],
                            gx.at[pl.ds(r0 * 128, k * 128)])

        start = cid * (nchunks // 2) + sid * ct

        def body(gi, _):
            group(start + gi * nb, nb)
            return 0
        lax.fori_loop(0, ng, body, 0)
        if tail:
            group(start + ng * nb, tail)

    return gather_k


@functools.cache
def _scatter_call(npad, oc, epad, first, split):
    nchunks = epad // 128
    ct = nchunks // 16 if split else nchunks // 32
    rows_per_tile = npad // 16
    wm = oc if split else 2 * oc
    zr = min(128, 16384 // wm)
    zch = rows_per_tile // zr

    sh_words = npad * wm + (npad * 16 if first else 0)
    per_tile_fixed = zr * wm + 8 * 128 + (2 * 128 * 16 if first else 0) + 1024
    avail = SPMEM_WORDS - sh_words - 16 * per_tile_fixed - 16384
    nb = max(1, min(8, avail // (16 * 128 * wm)))
    ng = ct // nb
    tail = ct - ng * nb

    out_type = [jax.ShapeDtypeStruct((npad, wm), F32)] * 2
    scratch = [
        pltpu.VMEM_SHARED((npad, wm), F32),
        pltpu.VMEM((zr, wm), F32),
        pltpu.VMEM((nb, 128), I32),
        pltpu.VMEM((nb * 128, wm), F32),
        pltpu.SemaphoreType.DMA,
        pltpu.SemaphoreType.DMA,
    ]
    if first:
        out_type += [jax.ShapeDtypeStruct((npad, 16), F32)] * (1 if split else 2)
        scratch.append(pltpu.VMEM_SHARED((npad, 16), F32))
        scratch.append(pltpu.VMEM((128, 16), F32))
        scratch.append(pltpu.VMEM((128, 16), F32))

    @functools.partial(
        pl.kernel,
        out_type=tuple(out_type),
        scratch_types=scratch,
        mesh=plsc.VectorSubcoreMesh(**_MESH),
        compiler_params=pltpu.CompilerParams(**_CPARAMS),
    )
    def scatter_k(*args):
        if split:
            msg0, msg1, dst2d = args[:3]
            rest = args[3:]
        else:
            msgb, dst2d = args[:2]
            rest = args[2:]
        agg0, agg1 = rest[:2]
        rest = rest[2:]
        if first:
            if split:
                (dego,) = rest[:1]
                rest = rest[1:]
            else:
                dego, deg1 = rest[:2]
                rest = rest[2:]
            sh_agg, zbuf, idx2, rows_v, sem, sem2, sh_deg, ones_v, zbuf16 = rest
        else:
            sh_agg, zbuf, idx2, rows_v, sem, sem2 = rest
        cid = lax.axis_index("c")
        sid = lax.axis_index("s")
        base = sid * rows_per_tile

        def zrow(r, _):
            for c in range(wm // 16):
                zbuf[r, pl.ds(c * 16, 16)] = jnp.zeros((16,), F32)
            if first and zr >= 128:
                ones_v[r] = jnp.ones((16,), F32)
                zbuf16[r] = jnp.zeros((16,), F32)
            return 0

        lax.fori_loop(0, zr, zrow, 0)
        if first and zr < 128:
            def orow(r, _):
                ones_v[r] = jnp.ones((16,), F32)
                zbuf16[r] = jnp.zeros((16,), F32)
                return 0
            lax.fori_loop(0, 128, orow, 0)

        descs = [pltpu.async_copy(zbuf, sh_agg.at[pl.ds(base + q * zr, zr)],
                                  sem)
                 for q in range(zch)]
        if first:
            def zdeg():
                dd = [pltpu.async_copy(
                    zbuf16, sh_deg.at[pl.ds(base + q * 128, 128)], sem2)
                    for q in range(rows_per_tile // 128)]
                for d in dd:
                    d.wait()
            if split:
                @pl.when(cid == 0)
                def _():
                    zdeg()
            else:
                zdeg()
        for d in descs:
            d.wait()
        plsc.subcore_barrier()

        def group(msg, r0, k, deg_too):
            din = [pltpu.async_copy(dst2d.at[pl.ds(r0, k)],
                                    idx2.at[pl.ds(0, k)], sem),
                   pltpu.async_copy(msg.at[pl.ds(r0 * 128, k * 128)],
                                    rows_v.at[pl.ds(0, k * 128)], sem)]
            for d in din:
                d.wait()
            descs = [pltpu.async_copy(rows_v.at[pl.ds(jb * 128, 128)],
                                      sh_agg.at[idx2.at[jb]], sem, add=True)
                     for jb in range(k)]
            if deg_too:
                dd = [pltpu.async_copy(ones_v, sh_deg.at[idx2.at[jb]],
                                       sem2, add=True)
                      for jb in range(k)]
                for d in dd:
                    d.wait()
            for d in descs:
                d.wait()

        def pipe(msg, start, deg_too):
            def body(gi, _):
                group(msg, start + gi * nb, nb, deg_too)
                return 0
            lax.fori_loop(0, ng, body, 0)
            if tail:
                group(msg, start + ng * nb, tail, deg_too)

        if split:
            @pl.when(cid == 0)
            def _():
                pipe(msg0, sid * ct, first)

            @pl.when(cid == 1)
            def _():
                pipe(msg1, sid * ct, False)
        else:
            start = cid * (nchunks // 2) + sid * ct
            pipe(msgb, start, first)

        plsc.subcore_barrier()

        def dump(agg, dg):
            dd = [pltpu.async_copy(sh_agg.at[pl.ds(base + q * zr, zr)],
                                   agg.at[pl.ds(base + q * zr, zr)], sem)
                  for q in range(zch)]
            if dg is not None:
                dd += [pltpu.async_copy(
                    sh_deg.at[pl.ds(base + q * 128, 128)],
                    dg.at[pl.ds(base + q * 128, 128)], sem2)
                    for q in range(rows_per_tile // 128)]
            for d in dd:
                d.wait()

        @pl.when(cid == 0)
        def _():
            dump(agg0, dego if first else None)

        @pl.when(cid == 1)
        def _():
            dump(agg1, (deg1 if (first and not split) else None))

    return scatter_k


@functools.cache
def _pool_call(npad_prev, f, npool):
    cn = 32 if f >= 128 else 64
    nq = cn * 8 // 128
    ch = npool // cn
    cpt = ch // 32
    lf = int(np.log2(f))
    w2 = 2 * f
    rows_words = cn * 8 * w2
    staged = False
    rows_per_tile = npad_prev // 16

    scratch = [
        pltpu.VMEM((nq, 128), I32),
        pltpu.VMEM((cn * 8, w2), F32),
        pltpu.VMEM((cn, w2), F32),
        pltpu.SemaphoreType.DMA,
    ]
    if staged:
        scratch.append(pltpu.VMEM_SHARED((npad_prev, w2), F32))

    @functools.partial(
        pl.kernel,
        out_type=jax.ShapeDtypeStruct((npool, w2), F32),
        scratch_types=scratch,
        mesh=plsc.VectorSubcoreMesh(**_MESH),
        compiler_params=pltpu.CompilerParams(**_CPARAMS,
                                             needs_layout_passes=False),
    )
    def pool_k(xb, hexidx, xp, idx_v, rows_v, out_v, sem, *sh):
        cid = lax.axis_index("c")
        sid = lax.axis_index("s")
        iot7 = 7 * lax.iota(I32, 16)
        if staged:
            sh_x = sh[0]
            base = sid * rows_per_tile
            pltpu.sync_copy(xb.at[pl.ds(base, rows_per_tile)],
                            sh_x.at[pl.ds(base, rows_per_tile)])
            plsc.subcore_barrier()
            src = sh_x
        else:
            src = xb

        def body(j, _):
            chn = cid * (ch // 2) + sid * cpt + j
            pltpu.sync_copy(hexidx.at[pl.ds(chn * nq, nq)], idx_v)
            dd = [pltpu.async_copy(
                src.at[idx_v.at[q]], rows_v.at[pl.ds(q * 128, 128)], sem)
                for q in range(nq)]
            for d in dd:
                d.wait()

            def node(i, _):
                for b in range(2):
                    for c in range(f // 16):
                        acc = None
                        for jj in range(7):
                            p = 112 * c + jj + iot7
                            row = 7 * i + (p >> lf)
                            col = b * f + (p & (f - 1))
                            v = plsc.load_gather(rows_v, [row, col])
                            acc = v if acc is None else jnp.maximum(acc, v)
                        out_v[i, pl.ds(b * f + c * 16, 16)] = acc
                return 0

            lax.fori_loop(0, cn, node, 0)
            pltpu.sync_copy(out_v, xp.at[pl.ds(chn * cn, cn)])
            return 0

        lax.fori_loop(0, cpt, body, 0)

    return pool_k


@functools.cache
def _conv_call(epad, ip, oc, be, split_out):
    ko = KP * oc

    def body(psd, gxb, mu, iv, g0, g1, expm, sel, *outs):
        p = psd[...]
        p0 = p[:, 0:1]
        p1 = p[:, 1:2]
        mua = mu[...]
        iva = iv[...]
        ea = expm[...]
        sa = sel[...]
        gxa = gxb[...]
        msgs = []
        for b in range(2):
            gx = gxa[:, b * ip:(b + 1) * ip]
            g = (g0, g1)[b][...]
            m0 = mua[2 * b:2 * b + 1, :]
            m1 = mua[2 * b + 1:2 * b + 2, :]
            i0 = iva[2 * b:2 * b + 1, :]
            i1 = iva[2 * b + 1:2 * b + 2, :]
            w = jnp.exp(-0.5 * ((p0 - m0) ** 2 * i0 + (p1 - m1) ** 2 * i1))
            wexp = jnp.dot(w, ea, preferred_element_type=F32)
            xj = jnp.dot(gx, g, preferred_element_type=F32)
            msgs.append(jnp.dot(xj * wexp, sa, preferred_element_type=F32))
        if split_out:
            outs[0][...] = msgs[0]
            outs[1][...] = msgs[1]
        else:
            outs[0][...] = jnp.concatenate(msgs, axis=1)

    const = lambda i: (0, 0)
    row = lambda i: (i, 0)
    if split_out:
        out_specs = [pl.BlockSpec((be, oc), row)] * 2
        out_shape = [jax.ShapeDtypeStruct((epad, oc), F32)] * 2
    else:
        out_specs = [pl.BlockSpec((be, 2 * oc), row)]
        out_shape = [jax.ShapeDtypeStruct((epad, 2 * oc), F32)]
    return pl.pallas_call(
        body,
        grid=(epad // be,),
        in_specs=[
            pl.BlockSpec((be, 2), row),
            pl.BlockSpec((be, 2 * ip), row),
            pl.BlockSpec((4, KP), const),
            pl.BlockSpec((4, KP), const),
            pl.BlockSpec((ip, ko), const),
            pl.BlockSpec((ip, ko), const),
            pl.BlockSpec((KP, ko), const),
            pl.BlockSpec((ko, oc), const),
        ],
        out_specs=out_specs,
        out_shape=out_shape,
    )


@functools.cache
def _root_call(npad, w, oc, bn=1024):
    def body(tb, r0, r1, bias, rt0, rt1):
        ba = bias[...]
        ta = tb[...]
        for b in range(2):
            t = ta[:, b * w:(b + 1) * w]
            r = (r0, r1)[b][...]
            (rt0, rt1)[b][...] = (jnp.dot(t, r, preferred_element_type=F32)
                                  + ba[b:b + 1, :])

    const = lambda i: (0, 0)
    row = lambda i: (i, 0)
    return pl.pallas_call(
        body,
        grid=(npad // bn,),
        in_specs=[
            pl.BlockSpec((bn, 2 * w), row),
            pl.BlockSpec((w, oc), const),
            pl.BlockSpec((w, oc), const),
            pl.BlockSpec((2, oc), const),
        ],
        out_specs=[pl.BlockSpec((bn, oc), row)] * 2,
        out_shape=[jax.ShapeDtypeStruct((npad, oc), F32)] * 2,
    )


@functools.cache
def _fin_call(npad, oc, ocn, split, bn=1024):
    mid = ocn is not None

    def body(a0, a1, *rest):
        if split:
            deg, rt0, rt1 = rest[:3]
            rest = rest[3:]
            d = jnp.maximum(deg[...][:, 0:1], 1.0)
            xs = [(a0, a1)[b][...] / d + (rt0, rt1)[b][...] for b in range(2)]
            xb = jnp.concatenate(xs, axis=1)
        else:
            deg0, deg1, rt0, rt1 = rest[:4]
            rest = rest[4:]
            d = jnp.maximum(deg0[...][:, 0:1] + deg1[...][:, 0:1], 1.0)
            s = a0[...] + a1[...]
            rtb = jnp.concatenate([rt0[...], rt1[...]], axis=1)
            xb = s / d + rtb
        xb = jnp.maximum(xb, 0.2 * xb)
        if mid:
            rn0, rn1, bnxt, xo, xr0, xr1 = rest
            xo[...] = xb
            for b in range(2):
                (xr0, xr1)[b][...] = (
                    jnp.dot(xb[:, b * oc:(b + 1) * oc], (rn0, rn1)[b][...],
                            preferred_element_type=F32)
                    + bnxt[...][b:b + 1, :])
        else:
            (xo,) = rest
            xo[...] = xb

    const = lambda i: (0, 0)
    row = lambda i: (i, 0)
    wa = oc if split else 2 * oc
    in_specs = [pl.BlockSpec((bn, wa), row), pl.BlockSpec((bn, wa), row)]
    in_specs += [pl.BlockSpec((bn, 16), row)] * (1 if split else 2)
    in_specs += [pl.BlockSpec((bn, oc), row)] * 2
    out_specs = [pl.BlockSpec((bn, 2 * oc), row)]
    out_shape = [jax.ShapeDtypeStruct((npad, 2 * oc), F32)]
    if mid:
        in_specs += ([pl.BlockSpec((oc, ocn), const)] * 2
                     + [pl.BlockSpec((2, ocn), const)])
        out_specs += [pl.BlockSpec((bn, ocn), row)] * 2
        out_shape += [jax.ShapeDtypeStruct((npad, ocn), F32)] * 2
    return pl.pallas_call(
        body,
        grid=(npad // bn,),
        in_specs=in_specs,
        out_specs=out_specs,
        out_shape=out_shape,
    )


@functools.cache
def _expand_sel(oc):
    e = np.zeros((KP, KP * oc), np.float32)
    s = np.zeros((KP * oc, oc), np.float32)
    for k in range(KP):
        e[k, k * oc:(k + 1) * oc] = 1.0
        s[k * oc:(k + 1) * oc, :] = np.eye(oc, dtype=np.float32)
    return jnp.asarray(e), jnp.asarray(s)


def _prep(params, name, inpad, oc):
    p = params[name]
    ic = p['g'].shape[0]
    g = jnp.zeros((inpad, KP * oc), F32).at[:ic, :NK * oc].set(p['g'])
    iv = 1.0 / (p['sigma'] ** 2 + EPS)
    mu0 = jnp.zeros((KP,), F32).at[:NK].set(p['mu'][:, 0])
    mu1 = jnp.zeros((KP,), F32).at[:NK].set(p['mu'][:, 1])
    iv0 = jnp.zeros((KP,), F32).at[:NK].set(iv[:, 0])
    iv1 = jnp.zeros((KP,), F32).at[:NK].set(iv[:, 1])
    root = jnp.zeros((inpad, oc), F32).at[:ic].set(p['root'])
    return g, mu0, mu1, iv0, iv1, root, p['bias']


def kernel(moving, target, edge_input, params,
           edge_index1, edge_index2, edge_index3, edge_index4,
           pseudo0, pseudo1, pseudo2, pseudo3, pseudo4,
           hex0, hex1, hex2, hex3):
    edges = [edge_input, edge_index1, edge_index2, edge_index3, edge_index4]
    pseudos = [pseudo0, pseudo1, pseudo2, pseudo3, pseudo4]
    hexes = [hex0, hex1, hex2, hex3]
    inp_b = [moving, target]

    src2d, dst2d, psd = [], [], []
    for l in range(5):
        e, ep = ES_[l], EPAD[l]
        s = jnp.zeros((ep,), I32).at[:e].set(edges[l][0])
        t = jnp.full((ep,), NS_[l], I32).at[:e].set(edges[l][1])
        src2d.append(s.reshape(ep // 128, 128))
        dst2d.append(t.reshape(ep // 128, 128))
        psd.append(jnp.zeros((ep, 2), F32).at[:e].set(pseudos[l]))

    hexidx = []
    for l in range(4):
        f = NF_[l]
        cn = 32 if f >= 128 else 64
        npl = _ru(NS_[l + 1], 2048)
        h = jnp.zeros((npl, 7), I32).at[:NS_[l + 1]].set(hexes[l])
        h = jnp.pad(h.reshape(npl // cn, cn * 7), ((0, 0), (0, cn)))
        hexidx.append(h.reshape(npl // cn * (cn * 8 // 128), 128))

    xb = jnp.zeros((NPAD[0], 32), F32)
    xb = xb.at[:NS_[0], 0:2].set(moving).at[:NS_[0], 16:18].set(target)
    rts = None

    for l in range(5):
        oc = NF_[l]
        in0 = 2 if l == 0 else 2 * NF_[l - 1] + 2
        inpads = [_ru(in0, 16), oc]
        split = SPLIT[l]
        names = [(X_NAMES[2 * l], Y_NAMES[2 * l]),
                 (X_NAMES[2 * l + 1], Y_NAMES[2 * l + 1])]
        W = [[_prep(params, names[j][b], inpads[j], oc) for b in range(2)]
             for j in range(2)]
        if l == 0:
            rts = _root_call(NPAD[0], 16, oc)(
                xb, W[0][0][5], W[0][1][5],
                jnp.stack([W[0][0][6], W[0][1][6]]))
        expm, sel = _expand_sel(oc)
        degs = None
        for j in (0, 1):
            wj = W[j]
            ip = inpads[j]
            gx = _gather_call(NPAD[l], 2 * ip, EPAD[l])(xb, src2d[l])
            mu = jnp.stack([wj[0][1], wj[0][2], wj[1][1], wj[1][2]])
            iv = jnp.stack([wj[0][3], wj[0][4], wj[1][3], wj[1][4]])
            msgs = _conv_call(EPAD[l], ip, oc, BE_[l], split)(
                psd[l], gx, mu, iv, wj[0][0], wj[1][0], expm, sel)
            sc = _scatter_call(NPAD[l], oc, EPAD[l], j == 0, split)
            if j == 0:
                if split:
                    a0, a1, dg = sc(msgs[0], msgs[1], dst2d[l])
                    degs = (dg,)
                else:
                    a0, a1, dg0, dg1 = sc(msgs[0], dst2d[l])
                    degs = (dg0, dg1)
                bnxt = jnp.stack([W[1][0][6], W[1][1][6]])
                outs = _fin_call(NPAD[l], oc, oc, split)(
                    a0, a1, *degs, rts[0], rts[1],
                    W[1][0][5], W[1][1][5], bnxt)
                xb, rts = outs[0], (outs[1], outs[2])
            else:
                if split:
                    a0, a1 = sc(msgs[0], msgs[1], dst2d[l])
                else:
                    a0, a1 = sc(msgs[0], dst2d[l])
                (xb,) = _fin_call(NPAD[l], oc, None, split)(
                    a0, a1, *degs, rts[0], rts[1])
        if l < 4:
            npl = _ru(NS_[l + 1], 2048)
            xp = _pool_call(NPAD[l], oc, npl)(xb, hexidx[l])
            dnew = NS_[l + 1]
            in_next = 2 * oc + 2
            ipn = _ru(in_next, 16)
            oc2 = NF_[l + 1]
            nm2 = (X_NAMES[2 * l + 2], Y_NAMES[2 * l + 2])
            Wn = [_prep(params, nm2[b], ipn, oc2) for b in range(2)]
            zpad = jnp.zeros((dnew, ipn - in_next), F32)
            halves = []
            for b in range(2):
                halves.append(jnp.concatenate(
                    [xb[:dnew, b * oc:(b + 1) * oc],
                     xp[:dnew, b * oc:(b + 1) * oc],
                     inp_b[b][:dnew], zpad], axis=1))
            tb = jnp.concatenate(halves, axis=1)
            xb = jnp.pad(tb, ((0, NPAD[l + 1] - dnew), (0, 0)))
            rts = _root_call(NPAD[l + 1], ipn, oc2)(
                xb, Wn[0][5], Wn[1][5], jnp.stack([Wn[0][6], Wn[1][6]]))
    return xb[:NS_[4], 0:256], xb[:NS_[4], 256:512]


# speedup vs baseline: 1.5023x; 1.5023x over previous
import functools

import jax
import jax.numpy as jnp
import numpy as np
from jax import lax
from jax.experimental import pallas as pl
from jax.experimental.pallas import tpu as pltpu
from jax.experimental.pallas import tpu_sc as plsc

F32 = jnp.float32
I32 = jnp.int32

NS_ = [40962, 10242, 2562, 642, 162]
ES_ = [6 * n for n in NS_]
NK = 10
KP = 16
NF_ = [16, 32, 64, 128, 256]
EPS = 1e-15
SPMEM_WORDS = 2097151


def _ru(x, m):
    return (x + m - 1) // m * m


EPAD = [_ru(e, 4096) for e in ES_]
NPAD = [_ru(n + 8, 2048) for n in NS_]
BE_ = [2048, 2048, 1024, 512, 256]

X_NAMES = ['conv1', 'conv1s', 'conv2', 'conv2s', 'conv3', 'conv3s',
           'conv4', 'conv4s', 'conv5', 'conv5s']
Y_NAMES = ['conv1_d', 'conv1s_d', 'conv2_d', 'conv2s_d', 'conv3_d',
           'conv3s_d', 'conv4', 'conv4s', 'conv5', 'conv5s']

_MESH = dict(core_axis_name="c", subcore_axis_name="s")


def _nbuf(w):
    return max(1, min(8, 376832 // (512 * w)))


@functools.cache
def _gather_call(npad, w, epad):
    nr_chunks = epad // 128
    ct = nr_chunks // 16
    rows_per_tile = npad // 16
    nb = _nbuf(w)
    ng = ct // nb
    tail = ct - ng * nb

    @functools.partial(
        pl.kernel,
        out_type=(jax.ShapeDtypeStruct((epad, w), F32),
                  jax.ShapeDtypeStruct((epad, w), F32)),
        scratch_types=[
            pltpu.VMEM_SHARED((npad, w), F32),
            pltpu.VMEM((nb, 128), I32),
            pltpu.VMEM((nb * 128, w), F32),
            pltpu.SemaphoreType.DMA,
        ],
        mesh=plsc.VectorSubcoreMesh(**_MESH),
        compiler_params=pltpu.CompilerParams(use_tc_tiling_on_sc=False),
    )
    def gather_k(x0, x1, src2d, gx0, gx1, sh_x, idx2, rows_v, sem):
        cid = lax.axis_index("c")
        sid = lax.axis_index("s")
        base = sid * rows_per_tile

        @pl.when(cid == 0)
        def _():
            pltpu.sync_copy(x0.at[pl.ds(base, rows_per_tile)],
                            sh_x.at[pl.ds(base, rows_per_tile)])

        @pl.when(cid == 1)
        def _():
            pltpu.sync_copy(x1.at[pl.ds(base, rows_per_tile)],
                            sh_x.at[pl.ds(base, rows_per_tile)])
        plsc.subcore_barrier()

        def group(gx, r0, k):
            pltpu.sync_copy(src2d.at[pl.ds(r0, k)], idx2.at[pl.ds(0, k)])
            descs = [pltpu.async_copy(sh_x.at[idx2.at[jb]],
                                      rows_v.at[pl.ds(jb * 128, 128)], sem)
                     for jb in range(k)]
            for d in descs:
                d.wait()
            pltpu.sync_copy(rows_v.at[pl.ds(0, k * 128)],
                            gx.at[pl.ds(r0 * 128, k * 128)])

        def pipe(gx):
            def body(gi, _):
                group(gx, sid * ct + gi * nb, nb)
                return 0
            lax.fori_loop(0, ng, body, 0)
            if tail:
                group(gx, sid * ct + ng * nb, tail)

        @pl.when(cid == 0)
        def _():
            pipe(gx0)

        @pl.when(cid == 1)
        def _():
            pipe(gx1)

    return gather_k


@functools.cache
def _scatter_call(npad, oc, epad, first):
    nr_chunks = epad // 128
    ct = nr_chunks // 16
    rows_per_tile = npad // 16
    zch = rows_per_tile // 128

    sh_words = npad * oc + (npad * 16 if first else 0)
    per_tile_fixed = 128 * oc + 8 * 128 + (2 * 128 * 16 if first else 0) + 1024
    avail = SPMEM_WORDS - sh_words - 16 * per_tile_fixed - 16384
    nb = max(1, min(8, avail // (16 * 128 * oc)))
    ng = ct // nb
    tail = ct - ng * nb

    out_type = [jax.ShapeDtypeStruct((npad, oc), F32),
                jax.ShapeDtypeStruct((npad, oc), F32)]
    scratch = [
        pltpu.VMEM_SHARED((npad, oc), F32),
        pltpu.VMEM((128, oc), F32),
        pltpu.VMEM((nb, 128), I32),
        pltpu.VMEM((nb * 128, oc), F32),
        pltpu.SemaphoreType.DMA,
        pltpu.SemaphoreType.DMA,
    ]
    if first:
        out_type.append(jax.ShapeDtypeStruct((npad, 16), F32))
        scratch.append(pltpu.VMEM_SHARED((npad, 16), F32))
        scratch.append(pltpu.VMEM((128, 16), F32))
        scratch.append(pltpu.VMEM((128, 16), F32))

    @functools.partial(
        pl.kernel,
        out_type=tuple(out_type),
        scratch_types=scratch,
        mesh=plsc.VectorSubcoreMesh(**_MESH),
        compiler_params=pltpu.CompilerParams(use_tc_tiling_on_sc=False),
    )
    def scatter_k(msg0, msg1, dst2d, agg0, agg1, *rest):
        if first:
            dego, sh_agg, zbuf, idx2, rows_v, sem, sem2, sh_deg, ones_v, zbuf16 = rest
        else:
            sh_agg, zbuf, idx2, rows_v, sem, sem2 = rest
        cid = lax.axis_index("c")
        sid = lax.axis_index("s")
        base = sid * rows_per_tile

        def zrow(r, _):
            for c in range(oc // 16):
                zbuf[r, pl.ds(c * 16, 16)] = jnp.zeros((16,), F32)
            if first:
                ones_v[r] = jnp.ones((16,), F32)
                zbuf16[r] = jnp.zeros((16,), F32)
            return 0

        lax.fori_loop(0, 128, zrow, 0)

        descs = [pltpu.async_copy(zbuf, sh_agg.at[pl.ds(base + q * 128, 128)],
                                  sem)
                 for q in range(zch)]
        if first:
            @pl.when(cid == 0)
            def _():
                dd = [pltpu.async_copy(
                    zbuf16, sh_deg.at[pl.ds(base + q * 128, 128)], sem2)
                    for q in range(zch)]
                for d in dd:
                    d.wait()
        for d in descs:
            d.wait()
        plsc.subcore_barrier()

        def group(msg, r0, k, deg_too):
            din = [pltpu.async_copy(dst2d.at[pl.ds(r0, k)],
                                    idx2.at[pl.ds(0, k)], sem),
                   pltpu.async_copy(msg.at[pl.ds(r0 * 128, k * 128)],
                                    rows_v.at[pl.ds(0, k * 128)], sem)]
            for d in din:
                d.wait()
            descs = [pltpu.async_copy(rows_v.at[pl.ds(jb * 128, 128)],
                                      sh_agg.at[idx2.at[jb]], sem, add=True)
                     for jb in range(k)]
            if deg_too:
                dd = [pltpu.async_copy(ones_v, sh_deg.at[idx2.at[jb]],
                                       sem2, add=True)
                      for jb in range(k)]
                for d in dd:
                    d.wait()
            for d in descs:
                d.wait()

        def pipe(msg, deg_too):
            def body(gi, _):
                group(msg, sid * ct + gi * nb, nb, deg_too)
                return 0
            lax.fori_loop(0, ng, body, 0)
            if tail:
                group(msg, sid * ct + ng * nb, tail, deg_too)

        @pl.when(cid == 0)
        def _():
            pipe(msg0, first)

        @pl.when(cid == 1)
        def _():
            pipe(msg1, False)

        plsc.subcore_barrier()

        @pl.when(cid == 0)
        def _():
            dd = [pltpu.async_copy(sh_agg.at[pl.ds(base + q * 128, 128)],
                                   agg0.at[pl.ds(base + q * 128, 128)], sem)
                  for q in range(zch)]
            if first:
                dd += [pltpu.async_copy(sh_deg.at[pl.ds(base + q * 128, 128)],
                                        dego.at[pl.ds(base + q * 128, 128)],
                                        sem2)
                       for q in range(zch)]
            for d in dd:
                d.wait()

        @pl.when(cid == 1)
        def _():
            dd = [pltpu.async_copy(sh_agg.at[pl.ds(base + q * 128, 128)],
                                   agg1.at[pl.ds(base + q * 128, 128)], sem)
                  for q in range(zch)]
            for d in dd:
                d.wait()

    return scatter_k


@functools.cache
def _pool_call(npad_prev, f, npool):
    ch = npool // 64
    cpt = ch // 16
    lf = int(np.log2(f))
    rows_per_tile = npad_prev // 16
    staged = (SPMEM_WORDS - npad_prev * f
              - 16 * (512 * f + 64 * f + 1024) - 16384) >= 0

    scratch = [
        pltpu.VMEM((4, 128), I32),
        pltpu.VMEM((512, f), F32),
        pltpu.VMEM((64, f), F32),
        pltpu.SemaphoreType.DMA,
    ]
    if staged:
        scratch.append(pltpu.VMEM_SHARED((npad_prev, f), F32))

    @functools.partial(
        pl.kernel,
        out_type=(jax.ShapeDtypeStruct((npool, f), F32),
                  jax.ShapeDtypeStruct((npool, f), F32)),
        scratch_types=scratch,
        mesh=plsc.VectorSubcoreMesh(**_MESH),
        compiler_params=pltpu.CompilerParams(use_tc_tiling_on_sc=False,
                                             needs_layout_passes=False),
    )
    def pool_k(x0, x1, hexidx, xp0, xp1, idx_v, rows_v, out_v, sem, *sh):
        cid = lax.axis_index("c")
        sid = lax.axis_index("s")
        iot7 = 7 * lax.iota(I32, 16)
        if staged:
            sh_x = sh[0]
            base = sid * rows_per_tile

            @pl.when(cid == 0)
            def _():
                pltpu.sync_copy(x0.at[pl.ds(base, rows_per_tile)],
                                sh_x.at[pl.ds(base, rows_per_tile)])

            @pl.when(cid == 1)
            def _():
                pltpu.sync_copy(x1.at[pl.ds(base, rows_per_tile)],
                                sh_x.at[pl.ds(base, rows_per_tile)])
            plsc.subcore_barrier()

        def body(j, _):
            chn = sid * cpt + j
            pltpu.sync_copy(hexidx.at[pl.ds(chn * 4, 4)], idx_v)

            def fetch(src):
                dd = [pltpu.async_copy(
                    src.at[idx_v.at[q]], rows_v.at[pl.ds(q * 128, 128)], sem)
                    for q in range(4)]
                for d in dd:
                    d.wait()

            if staged:
                fetch(sh_x)
            else:
                @pl.when(cid == 0)
                def _():
                    fetch(x0)

                @pl.when(cid == 1)
                def _():
                    fetch(x1)

            def node(i, _):
                for c in range(f // 16):
                    acc = None
                    for jj in range(7):
                        p = 112 * c + jj + iot7
                        row = 7 * i + (p >> lf)
                        col = p & (f - 1)
                        v = plsc.load_gather(rows_v, [row, col])
                        acc = v if acc is None else jnp.maximum(acc, v)
                    out_v[i, pl.ds(c * 16, 16)] = acc
                return 0

            lax.fori_loop(0, 64, node, 0)

            @pl.when(cid == 0)
            def _():
                pltpu.sync_copy(out_v, xp0.at[pl.ds(chn * 64, 64)])

            @pl.when(cid == 1)
            def _():
                pltpu.sync_copy(out_v, xp1.at[pl.ds(chn * 64, 64)])
            return 0

        lax.fori_loop(0, cpt, body, 0)

    return pool_k


@functools.cache
def _conv_call(epad, inpad, oc, be):
    ko = KP * oc

    def body(psd, gx0, gx1, mu, iv, g0, g1, expm, sel, msg0, msg1):
        p = psd[...]
        p0 = p[:, 0:1]
        p1 = p[:, 1:2]
        mua = mu[...]
        iva = iv[...]
        ea = expm[...]
        sa = sel[...]
        for b in range(2):
            gx = (gx0, gx1)[b][...]
            g = (g0, g1)[b][...]
            m0 = mua[2 * b:2 * b + 1, :]
            m1 = mua[2 * b + 1:2 * b + 2, :]
            i0 = iva[2 * b:2 * b + 1, :]
            i1 = iva[2 * b + 1:2 * b + 2, :]
            w = jnp.exp(-0.5 * ((p0 - m0) ** 2 * i0 + (p1 - m1) ** 2 * i1))
            wexp = jnp.dot(w, ea, preferred_element_type=F32)
            xj = jnp.dot(gx, g, preferred_element_type=F32)
            (msg0, msg1)[b][...] = jnp.dot(xj * wexp, sa,
                                           preferred_element_type=F32)

    const = lambda i: (0, 0)
    row = lambda i: (i, 0)
    return pl.pallas_call(
        body,
        grid=(epad // be,),
        in_specs=[
            pl.BlockSpec((be, 2), row),
            pl.BlockSpec((be, inpad), row),
            pl.BlockSpec((be, inpad), row),
            pl.BlockSpec((4, KP), const),
            pl.BlockSpec((4, KP), const),
            pl.BlockSpec((inpad, ko), const),
            pl.BlockSpec((inpad, ko), const),
            pl.BlockSpec((KP, ko), const),
            pl.BlockSpec((ko, oc), const),
        ],
        out_specs=[pl.BlockSpec((be, oc), row)] * 2,
        out_shape=[jax.ShapeDtypeStruct((epad, oc), F32)] * 2,
    )


@functools.cache
def _root_call(npad, w, oc, bn=1024):
    def body(t0, t1, r0, r1, bias, rt0, rt1):
        ba = bias[...]
        for b in range(2):
            t = (t0, t1)[b][...]
            r = (r0, r1)[b][...]
            (rt0, rt1)[b][...] = (jnp.dot(t, r, preferred_element_type=F32)
                                  + ba[b:b + 1, :])

    const = lambda i: (0, 0)
    row = lambda i: (i, 0)
    return pl.pallas_call(
        body,
        grid=(npad // bn,),
        in_specs=[
            pl.BlockSpec((bn, w), row),
            pl.BlockSpec((bn, w), row),
            pl.BlockSpec((w, oc), const),
            pl.BlockSpec((w, oc), const),
            pl.BlockSpec((2, oc), const),
        ],
        out_specs=[pl.BlockSpec((bn, oc), row)] * 2,
        out_shape=[jax.ShapeDtypeStruct((npad, oc), F32)] * 2,
    )


@functools.cache
def _fin_call(npad, oc, ocn, bn=1024):
    mid = ocn is not None

    def body(a0, a1, deg, rt0, rt1, *rest):
        if mid:
            rn0, rn1, bnxt, x0, x1, xr0, xr1 = rest
        else:
            x0, x1 = rest
        d = jnp.maximum(deg[...][:, 0:1], 1.0)
        for b in range(2):
            x = (a0, a1)[b][...] / d + (rt0, rt1)[b][...]
            x = jnp.maximum(x, 0.2 * x)
            (x0, x1)[b][...] = x
            if mid:
                (xr0, xr1)[b][...] = (
                    jnp.dot(x, (rn0, rn1)[b][...], preferred_element_type=F32)
                    + bnxt[...][b:b + 1, :])

    const = lambda i: (0, 0)
    row = lambda i: (i, 0)
    in_specs = [
        pl.BlockSpec((bn, oc), row),
        pl.BlockSpec((bn, oc), row),
        pl.BlockSpec((bn, 16), row),
        pl.BlockSpec((bn, oc), row),
        pl.BlockSpec((bn, oc), row),
    ]
    out_specs = [pl.BlockSpec((bn, oc), row)] * 2
    out_shape = [jax.ShapeDtypeStruct((npad, oc), F32)] * 2
    if mid:
        in_specs += ([pl.BlockSpec((oc, ocn), const)] * 2
                     + [pl.BlockSpec((2, ocn), const)])
        out_specs += [pl.BlockSpec((bn, ocn), row)] * 2
        out_shape += [jax.ShapeDtypeStruct((npad, ocn), F32)] * 2
    return pl.pallas_call(
        body,
        grid=(npad // bn,),
        in_specs=in_specs,
        out_specs=out_specs,
        out_shape=out_shape,
    )


@functools.cache
def _expand_sel(oc):
    e = np.zeros((KP, KP * oc), np.float32)
    s = np.zeros((KP * oc, oc), np.float32)
    for k in range(KP):
        e[k, k * oc:(k + 1) * oc] = 1.0
        s[k * oc:(k + 1) * oc, :] = np.eye(oc, dtype=np.float32)
    return jnp.asarray(e), jnp.asarray(s)


def _prep(params, name, inpad, oc):
    p = params[name]
    ic = p['g'].shape[0]
    g = jnp.zeros((inpad, KP * oc), F32).at[:ic, :NK * oc].set(p['g'])
    iv = 1.0 / (p['sigma'] ** 2 + EPS)
    mu0 = jnp.zeros((KP,), F32).at[:NK].set(p['mu'][:, 0])
    mu1 = jnp.zeros((KP,), F32).at[:NK].set(p['mu'][:, 1])
    iv0 = jnp.zeros((KP,), F32).at[:NK].set(iv[:, 0])
    iv1 = jnp.zeros((KP,), F32).at[:NK].set(iv[:, 1])
    root = jnp.zeros((inpad, oc), F32).at[:ic].set(p['root'])
    return g, mu0, mu1, iv0, iv1, root, p['bias']


def kernel(moving, target, edge_input, params,
           edge_index1, edge_index2, edge_index3, edge_index4,
           pseudo0, pseudo1, pseudo2, pseudo3, pseudo4,
           hex0, hex1, hex2, hex3):
    edges = [edge_input, edge_index1, edge_index2, edge_index3, edge_index4]
    pseudos = [pseudo0, pseudo1, pseudo2, pseudo3, pseudo4]
    hexes = [hex0, hex1, hex2, hex3]
    inp_b = [moving, target]

    src2d, dst2d, psd = [], [], []
    for l in range(5):
        e, ep = ES_[l], EPAD[l]
        s = jnp.zeros((ep,), I32).at[:e].set(edges[l][0])
        t = jnp.full((ep,), NS_[l], I32).at[:e].set(edges[l][1])
        src2d.append(s.reshape(ep // 128, 128))
        dst2d.append(t.reshape(ep // 128, 128))
        psd.append(jnp.zeros((ep, 2), F32).at[:e].set(pseudos[l]))

    hexidx = []
    for l in range(4):
        npl = _ru(NS_[l + 1], 1024)
        h = jnp.zeros((npl, 7), I32).at[:NS_[l + 1]].set(hexes[l])
        h = jnp.pad(h.reshape(npl // 64, 448), ((0, 0), (0, 64)))
        hexidx.append(h.reshape(npl // 64 * 4, 128))

    tbls = [jnp.zeros((NPAD[0], 16), F32).at[:NS_[0], :2].set(inp_b[b])
            for b in range(2)]
    rts = None

    for l in range(5):
        oc = NF_[l]
        in0 = 2 if l == 0 else 2 * NF_[l - 1] + 2
        inpads = [_ru(in0, 16), oc]
        names = [(X_NAMES[2 * l], Y_NAMES[2 * l]),
                 (X_NAMES[2 * l + 1], Y_NAMES[2 * l + 1])]
        W = [[_prep(params, names[j][b], inpads[j], oc) for b in range(2)]
             for j in range(2)]
        if l == 0:
            rts = _root_call(NPAD[0], 16, oc)(
                tbls[0], tbls[1], W[0][0][5], W[0][1][5],
                jnp.stack([W[0][0][6], W[0][1][6]]))
        expm, sel = _expand_sel(oc)
        deg = None
        for j in (0, 1):
            wj = W[j]
            ip = inpads[j]
            gx0, gx1 = _gather_call(NPAD[l], ip, EPAD[l])(
                tbls[0], tbls[1], src2d[l])
            mu = jnp.stack([wj[0][1], wj[0][2], wj[1][1], wj[1][2]])
            iv = jnp.stack([wj[0][3], wj[0][4], wj[1][3], wj[1][4]])
            msg0, msg1 = _conv_call(EPAD[l], ip, oc, BE_[l])(
                psd[l], gx0, gx1, mu, iv, wj[0][0], wj[1][0], expm, sel)
            if j == 0:
                agg0, agg1, deg = _scatter_call(NPAD[l], oc, EPAD[l], True)(
                    msg0, msg1, dst2d[l])
                bnxt = jnp.stack([W[1][0][6], W[1][1][6]])
                x0, x1, rt0, rt1 = _fin_call(NPAD[l], oc, oc)(
                    agg0, agg1, deg, rts[0], rts[1],
                    W[1][0][5], W[1][1][5], bnxt)
                tbls = [x0, x1]
                rts = (rt0, rt1)
            else:
                agg0, agg1 = _scatter_call(NPAD[l], oc, EPAD[l], False)(
                    msg0, msg1, dst2d[l])
                x0, x1 = _fin_call(NPAD[l], oc, None)(
                    agg0, agg1, deg, rts[0], rts[1])
                tbls = [x0, x1]
        if l < 4:
            npl = _ru(NS_[l + 1], 1024)
            xp0, xp1 = _pool_call(NPAD[l], oc, npl)(tbls[0], tbls[1], hexidx[l])
            dnew = NS_[l + 1]
            in_next = 2 * oc + 2
            ipn = _ru(in_next, 16)
            oc2 = NF_[l + 1]
            nm2 = (X_NAMES[2 * l + 2], Y_NAMES[2 * l + 2])
            Wn = [_prep(params, nm2[b], ipn, oc2) for b in range(2)]
            newt = []
            for b in range(2):
                t = jnp.concatenate(
                    [tbls[b][:dnew, :oc], (xp0, xp1)[b][:dnew],
                     inp_b[b][:dnew]], axis=1)
                t = jnp.pad(t, ((0, NPAD[l + 1] - dnew), (0, ipn - in_next)))
                newt.append(t)
            tbls = newt
            rts = _root_call(NPAD[l + 1], ipn, oc2)(
                tbls[0], tbls[1], Wn[0][5], Wn[1][5],
                jnp.stack([Wn[0][6], Wn[1][6]]))
    return tbls[0][:NS_[4]], tbls[1][:NS_[4]]


# speedup vs baseline: 1.5387x; 1.0242x over previous
import functools

import jax
import jax.numpy as jnp
import numpy as np
from jax import lax
from jax.experimental import pallas as pl
from jax.experimental.pallas import tpu as pltpu
from jax.experimental.pallas import tpu_sc as plsc

F32 = jnp.float32
I32 = jnp.int32

NS_ = [40962, 10242, 2562, 642, 162]
ES_ = [6 * n for n in NS_]
NK = 10
KP = 16
NF_ = [16, 32, 64, 128, 256]
EPS = 1e-15
SPMEM_WORDS = 2097151


def _ru(x, m):
    return (x + m - 1) // m * m


EPAD = [_ru(e, 4096) for e in ES_]
NPAD = [_ru(n + 8, 2048) for n in NS_]
BE_ = [4096, 4096, 2048, 512, 256]

X_NAMES = ['conv1', 'conv1s', 'conv2', 'conv2s', 'conv3', 'conv3s',
           'conv4', 'conv4s', 'conv5', 'conv5s']
Y_NAMES = ['conv1_d', 'conv1s_d', 'conv2_d', 'conv2s_d', 'conv3_d',
           'conv3s_d', 'conv4', 'conv4s', 'conv5', 'conv5s']

_MESH = dict(core_axis_name="c", subcore_axis_name="s")


def _nbuf(w):
    return max(1, min(8, 376832 // (512 * w)))


@functools.cache
def _gather_call(npad, w, epad):
    nr_chunks = epad // 128
    ct = nr_chunks // 16
    rows_per_tile = npad // 16
    nb = _nbuf(w)
    ng = ct // nb
    tail = ct - ng * nb

    @functools.partial(
        pl.kernel,
        out_type=(jax.ShapeDtypeStruct((epad, w), F32),
                  jax.ShapeDtypeStruct((epad, w), F32)),
        scratch_types=[
            pltpu.VMEM_SHARED((npad, w), F32),
            pltpu.VMEM((nb, 128), I32),
            pltpu.VMEM((nb * 128, w), F32),
            pltpu.SemaphoreType.DMA,
        ],
        mesh=plsc.VectorSubcoreMesh(**_MESH),
        compiler_params=pltpu.CompilerParams(use_tc_tiling_on_sc=False),
    )
    def gather_k(x0, x1, src2d, gx0, gx1, sh_x, idx2, rows_v, sem):
        cid = lax.axis_index("c")
        sid = lax.axis_index("s")
        base = sid * rows_per_tile

        @pl.when(cid == 0)
        def _():
            pltpu.sync_copy(x0.at[pl.ds(base, rows_per_tile)],
                            sh_x.at[pl.ds(base, rows_per_tile)])

        @pl.when(cid == 1)
        def _():
            pltpu.sync_copy(x1.at[pl.ds(base, rows_per_tile)],
                            sh_x.at[pl.ds(base, rows_per_tile)])
        plsc.subcore_barrier()

        def group(gx, r0, k):
            pltpu.sync_copy(src2d.at[pl.ds(r0, k)], idx2.at[pl.ds(0, k)])
            descs = [pltpu.async_copy(sh_x.at[idx2.at[jb]],
                                      rows_v.at[pl.ds(jb * 128, 128)], sem)
                     for jb in range(k)]
            for d in descs:
                d.wait()
            pltpu.sync_copy(rows_v.at[pl.ds(0, k * 128)],
                            gx.at[pl.ds(r0 * 128, k * 128)])

        def pipe(gx):
            def body(gi, _):
                group(gx, sid * ct + gi * nb, nb)
                return 0
            lax.fori_loop(0, ng, body, 0)
            if tail:
                group(gx, sid * ct + ng * nb, tail)

        @pl.when(cid == 0)
        def _():
            pipe(gx0)

        @pl.when(cid == 1)
        def _():
            pipe(gx1)

    return gather_k


@functools.cache
def _scatter_call(npad, oc, epad, first):
    nr_chunks = epad // 128
    ct = nr_chunks // 16
    rows_per_tile = npad // 16
    zch = rows_per_tile // 128

    sh_words = npad * oc + (npad * 16 if first else 0)
    per_tile_fixed = 128 * oc + 8 * 128 + (2 * 128 * 16 if first else 0) + 1024
    avail = SPMEM_WORDS - sh_words - 16 * per_tile_fixed - 16384
    nb = max(1, min(8, avail // (16 * 128 * oc)))
    ng = ct // nb
    tail = ct - ng * nb

    out_type = [jax.ShapeDtypeStruct((npad, oc), F32),
                jax.ShapeDtypeStruct((npad, oc), F32)]
    scratch = [
        pltpu.VMEM_SHARED((npad, oc), F32),
        pltpu.VMEM((128, oc), F32),
        pltpu.VMEM((nb, 128), I32),
        pltpu.VMEM((nb * 128, oc), F32),
        pltpu.SemaphoreType.DMA,
        pltpu.SemaphoreType.DMA,
    ]
    if first:
        out_type.append(jax.ShapeDtypeStruct((npad, 16), F32))
        scratch.append(pltpu.VMEM_SHARED((npad, 16), F32))
        scratch.append(pltpu.VMEM((128, 16), F32))
        scratch.append(pltpu.VMEM((128, 16), F32))

    @functools.partial(
        pl.kernel,
        out_type=tuple(out_type),
        scratch_types=scratch,
        mesh=plsc.VectorSubcoreMesh(**_MESH),
        compiler_params=pltpu.CompilerParams(use_tc_tiling_on_sc=False),
    )
    def scatter_k(msg0, msg1, dst2d, agg0, agg1, *rest):
        if first:
            dego, sh_agg, zbuf, idx2, rows_v, sem, sem2, sh_deg, ones_v, zbuf16 = rest
        else:
            sh_agg, zbuf, idx2, rows_v, sem, sem2 = rest
        cid = lax.axis_index("c")
        sid = lax.axis_index("s")
        base = sid * rows_per_tile

        def zrow(r, _):
            for c in range(oc // 16):
                zbuf[r, pl.ds(c * 16, 16)] = jnp.zeros((16,), F32)
            if first:
                ones_v[r] = jnp.ones((16,), F32)
                zbuf16[r] = jnp.zeros((16,), F32)
            return 0

        lax.fori_loop(0, 128, zrow, 0)

        descs = [pltpu.async_copy(zbuf, sh_agg.at[pl.ds(base + q * 128, 128)],
                                  sem)
                 for q in range(zch)]
        if first:
            @pl.when(cid == 0)
            def _():
                dd = [pltpu.async_copy(
                    zbuf16, sh_deg.at[pl.ds(base + q * 128, 128)], sem2)
                    for q in range(zch)]
                for d in dd:
                    d.wait()
        for d in descs:
            d.wait()
        plsc.subcore_barrier()

        def group(msg, r0, k, deg_too):
            din = [pltpu.async_copy(dst2d.at[pl.ds(r0, k)],
                                    idx2.at[pl.ds(0, k)], sem),
                   pltpu.async_copy(msg.at[pl.ds(r0 * 128, k * 128)],
                                    rows_v.at[pl.ds(0, k * 128)], sem)]
            for d in din:
                d.wait()
            descs = [pltpu.async_copy(rows_v.at[pl.ds(jb * 128, 128)],
                                      sh_agg.at[idx2.at[jb]], sem, add=True)
                     for jb in range(k)]
            if deg_too:
                dd = [pltpu.async_copy(ones_v, sh_deg.at[idx2.at[jb]],
                                       sem2, add=True)
                      for jb in range(k)]
                for d in dd:
                    d.wait()
            for d in descs:
                d.wait()

        def pipe(msg, deg_too):
            def body(gi, _):
                group(msg, sid * ct + gi * nb, nb, deg_too)
                return 0
            lax.fori_loop(0, ng, body, 0)
            if tail:
                group(msg, sid * ct + ng * nb, tail, deg_too)

        @pl.when(cid == 0)
        def _():
            pipe(msg0, first)

        @pl.when(cid == 1)
        def _():
            pipe(msg1, False)

        plsc.subcore_barrier()

        @pl.when(cid == 0)
        def _():
            dd = [pltpu.async_copy(sh_agg.at[pl.ds(base + q * 128, 128)],
                                   agg0.at[pl.ds(base + q * 128, 128)], sem)
                  for q in range(zch)]
            if first:
                dd += [pltpu.async_copy(sh_deg.at[pl.ds(base + q * 128, 128)],
                                        dego.at[pl.ds(base + q * 128, 128)],
                                        sem2)
                       for q in range(zch)]
            for d in dd:
                d.wait()

        @pl.when(cid == 1)
        def _():
            dd = [pltpu.async_copy(sh_agg.at[pl.ds(base + q * 128, 128)],
                                   agg1.at[pl.ds(base + q * 128, 128)], sem)
                  for q in range(zch)]
            for d in dd:
                d.wait()

    return scatter_k


@functools.cache
def _pool_call(npad_prev, f, npool):
    ch = npool // 64
    cpt = ch // 16
    lf = int(np.log2(f))
    rows_per_tile = npad_prev // 16
    staged = (SPMEM_WORDS - npad_prev * f
              - 16 * (512 * f + 64 * f + 1024) - 16384) >= 0

    scratch = [
        pltpu.VMEM((4, 128), I32),
        pltpu.VMEM((512, f), F32),
        pltpu.VMEM((64, f), F32),
        pltpu.SemaphoreType.DMA,
    ]
    if staged:
        scratch.append(pltpu.VMEM_SHARED((npad_prev, f), F32))

    @functools.partial(
        pl.kernel,
        out_type=(jax.ShapeDtypeStruct((npool, f), F32),
                  jax.ShapeDtypeStruct((npool, f), F32)),
        scratch_types=scratch,
        mesh=plsc.VectorSubcoreMesh(**_MESH),
        compiler_params=pltpu.CompilerParams(use_tc_tiling_on_sc=False,
                                             needs_layout_passes=False),
    )
    def pool_k(x0, x1, hexidx, xp0, xp1, idx_v, rows_v, out_v, sem, *sh):
        cid = lax.axis_index("c")
        sid = lax.axis_index("s")
        iot7 = 7 * lax.iota(I32, 16)
        if staged:
            sh_x = sh[0]
            base = sid * rows_per_tile

            @pl.when(cid == 0)
            def _():
                pltpu.sync_copy(x0.at[pl.ds(base, rows_per_tile)],
                                sh_x.at[pl.ds(base, rows_per_tile)])

            @pl.when(cid == 1)
            def _():
                pltpu.sync_copy(x1.at[pl.ds(base, rows_per_tile)],
                                sh_x.at[pl.ds(base, rows_per_tile)])
            plsc.subcore_barrier()

        def body(j, _):
            chn = sid * cpt + j
            pltpu.sync_copy(hexidx.at[pl.ds(chn * 4, 4)], idx_v)

            def fetch(src):
                dd = [pltpu.async_copy(
                    src.at[idx_v.at[q]], rows_v.at[pl.ds(q * 128, 128)], sem)
                    for q in range(4)]
                for d in dd:
                    d.wait()

            if staged:
                fetch(sh_x)
            else:
                @pl.when(cid == 0)
                def _():
                    fetch(x0)

                @pl.when(cid == 1)
                def _():
                    fetch(x1)

            def node(i, _):
                for c in range(f // 16):
                    acc = None
                    for jj in range(7):
                        p = 112 * c + jj + iot7
                        row = 7 * i + (p >> lf)
                        col = p & (f - 1)
                        v = plsc.load_gather(rows_v, [row, col])
                        acc = v if acc is None else jnp.maximum(acc, v)
                    out_v[i, pl.ds(c * 16, 16)] = acc
                return 0

            lax.fori_loop(0, 64, node, 0)

            @pl.when(cid == 0)
            def _():
                pltpu.sync_copy(out_v, xp0.at[pl.ds(chn * 64, 64)])

            @pl.when(cid == 1)
            def _():
                pltpu.sync_copy(out_v, xp1.at[pl.ds(chn * 64, 64)])
            return 0

        lax.fori_loop(0, cpt, body, 0)

    return pool_k


@functools.cache
def _conv_call(epad, inpad, oc, be):
    ko = KP * oc

    def body(psd, gx0, gx1, mu, iv, g0, g1, expm, sel, msg0, msg1):
        p = psd[...]
        p0 = p[:, 0:1]
        p1 = p[:, 1:2]
        mua = mu[...]
        iva = iv[...]
        ea = expm[...]
        sa = sel[...]
        for b in range(2):
            gx = (gx0, gx1)[b][...]
            g = (g0, g1)[b][...]
            m0 = mua[2 * b:2 * b + 1, :]
            m1 = mua[2 * b + 1:2 * b + 2, :]
            i0 = iva[2 * b:2 * b + 1, :]
            i1 = iva[2 * b + 1:2 * b + 2, :]
            w = jnp.exp(-0.5 * ((p0 - m0) ** 2 * i0 + (p1 - m1) ** 2 * i1))
            wexp = jnp.dot(w, ea, preferred_element_type=F32)
            xj = jnp.dot(gx, g, preferred_element_type=F32)
            (msg0, msg1)[b][...] = jnp.dot(xj * wexp, sa,
                                           preferred_element_type=F32)

    const = lambda i: (0, 0)
    row = lambda i: (i, 0)
    return pl.pallas_call(
        body,
        grid=(epad // be,),
        in_specs=[
            pl.BlockSpec((be, 2), row),
            pl.BlockSpec((be, inpad), row),
            pl.BlockSpec((be, inpad), row),
            pl.BlockSpec((4, KP), const),
            pl.BlockSpec((4, KP), const),
            pl.BlockSpec((inpad, ko), const),
            pl.BlockSpec((inpad, ko), const),
            pl.BlockSpec((KP, ko), const),
            pl.BlockSpec((ko, oc), const),
        ],
        out_specs=[pl.BlockSpec((be, oc), row)] * 2,
        out_shape=[jax.ShapeDtypeStruct((epad, oc), F32)] * 2,
    )


@functools.cache
def _root_call(npad, w, oc, bn=1024):
    def body(t0, t1, r0, r1, bias, rt0, rt1):
        ba = bias[...]
        for b in range(2):
            t = (t0, t1)[b][...]
            r = (r0, r1)[b][...]
            (rt0, rt1)[b][...] = (jnp.dot(t, r, preferred_element_type=F32)
                                  + ba[b:b + 1, :])

    const = lambda i: (0, 0)
    row = lambda i: (i, 0)
    return pl.pallas_call(
        body,
        grid=(npad // bn,),
        in_specs=[
            pl.BlockSpec((bn, w), row),
            pl.BlockSpec((bn, w), row),
            pl.BlockSpec((w, oc), const),
            pl.BlockSpec((w, oc), const),
            pl.BlockSpec((2, oc), const),
        ],
        out_specs=[pl.BlockSpec((bn, oc), row)] * 2,
        out_shape=[jax.ShapeDtypeStruct((npad, oc), F32)] * 2,
    )


@functools.cache
def _fin_call(npad, oc, ocn, bn=1024):
    mid = ocn is not None

    def body(a0, a1, deg, rt0, rt1, *rest):
        if mid:
            rn0, rn1, bnxt, x0, x1, xr0, xr1 = rest
        else:
            x0, x1 = rest
        d = jnp.maximum(deg[...][:, 0:1], 1.0)
        for b in range(2):
            x = (a0, a1)[b][...] / d + (rt0, rt1)[b][...]
            x = jnp.maximum(x, 0.2 * x)
            (x0, x1)[b][...] = x
            if mid:
                (xr0, xr1)[b][...] = (
                    jnp.dot(x, (rn0, rn1)[b][...], preferred_element_type=F32)
                    + bnxt[...][b:b + 1, :])

    const = lambda i: (0, 0)
    row = lambda i: (i, 0)
    in_specs = [
        pl.BlockSpec((bn, oc), row),
        pl.BlockSpec((bn, oc), row),
        pl.BlockSpec((bn, 16), row),
        pl.BlockSpec((bn, oc), row),
        pl.BlockSpec((bn, oc), row),
    ]
    out_specs = [pl.BlockSpec((bn, oc), row)] * 2
    out_shape = [jax.ShapeDtypeStruct((npad, oc), F32)] * 2
    if mid:
        in_specs += ([pl.BlockSpec((oc, ocn), const)] * 2
                     + [pl.BlockSpec((2, ocn), const)])
        out_specs += [pl.BlockSpec((bn, ocn), row)] * 2
        out_shape += [jax.ShapeDtypeStruct((npad, ocn), F32)] * 2
    return pl.pallas_call(
        body,
        grid=(npad // bn,),
        in_specs=in_specs,
        out_specs=out_specs,
        out_shape=out_shape,
    )


@functools.cache
def _expand_sel(oc):
    e = np.zeros((KP, KP * oc), np.float32)
    s = np.zeros((KP * oc, oc), np.float32)
    for k in range(KP):
        e[k, k * oc:(k + 1) * oc] = 1.0
        s[k * oc:(k + 1) * oc, :] = np.eye(oc, dtype=np.float32)
    return jnp.asarray(e), jnp.asarray(s)


def _prep(params, name, inpad, oc):
    p = params[name]
    ic = p['g'].shape[0]
    g = jnp.zeros((inpad, KP * oc), F32).at[:ic, :NK * oc].set(p['g'])
    iv = 1.0 / (p['sigma'] ** 2 + EPS)
    mu0 = jnp.zeros((KP,), F32).at[:NK].set(p['mu'][:, 0])
    mu1 = jnp.zeros((KP,), F32).at[:NK].set(p['mu'][:, 1])
    iv0 = jnp.zeros((KP,), F32).at[:NK].set(iv[:, 0])
    iv1 = jnp.zeros((KP,), F32).at[:NK].set(iv[:, 1])
    root = jnp.zeros((inpad, oc), F32).at[:ic].set(p['root'])
    return g, mu0, mu1, iv0, iv1, root, p['bias']


def kernel(moving, target, edge_input, params,
           edge_index1, edge_index2, edge_index3, edge_index4,
           pseudo0, pseudo1, pseudo2, pseudo3, pseudo4,
           hex0, hex1, hex2, hex3):
    edges = [edge_input, edge_index1, edge_index2, edge_index3, edge_index4]
    pseudos = [pseudo0, pseudo1, pseudo2, pseudo3, pseudo4]
    hexes = [hex0, hex1, hex2, hex3]
    inp_b = [moving, target]

    src2d, dst2d, psd = [], [], []
    for l in range(5):
        e, ep = ES_[l], EPAD[l]
        s = jnp.zeros((ep,), I32).at[:e].set(edges[l][0])
        t = jnp.full((ep,), NS_[l], I32).at[:e].set(edges[l][1])
        src2d.append(s.reshape(ep // 128, 128))
        dst2d.append(t.reshape(ep // 128, 128))
        psd.append(jnp.zeros((ep, 2), F32).at[:e].set(pseudos[l]))

    hexidx = []
    for l in range(4):
        npl = _ru(NS_[l + 1], 1024)
        h = jnp.zeros((npl, 7), I32).at[:NS_[l + 1]].set(hexes[l])
        h = jnp.pad(h.reshape(npl // 64, 448), ((0, 0), (0, 64)))
        hexidx.append(h.reshape(npl // 64 * 4, 128))

    tbls = [jnp.zeros((NPAD[0], 16), F32).at[:NS_[0], :2].set(inp_b[b])
            for b in range(2)]
    rts = None

    for l in range(5):
        oc = NF_[l]
        in0 = 2 if l == 0 else 2 * NF_[l - 1] + 2
        inpads = [_ru(in0, 16), oc]
        names = [(X_NAMES[2 * l], Y_NAMES[2 * l]),
                 (X_NAMES[2 * l + 1], Y_NAMES[2 * l + 1])]
        W = [[_prep(params, names[j][b], inpads[j], oc) for b in range(2)]
             for j in range(2)]
        if l == 0:
            rts = _root_call(NPAD[0], 16, oc)(
                tbls[0], tbls[1], W[0][0][5], W[0][1][5],
                jnp.stack([W[0][0][6], W[0][1][6]]))
        expm, sel = _expand_sel(oc)
        deg = None
        for j in (0, 1):
            wj = W[j]
            ip = inpads[j]
            gx0, gx1 = _gather_call(NPAD[l], ip, EPAD[l])(
                tbls[0], tbls[1], src2d[l])
            mu = jnp.stack([wj[0][1], wj[0][2], wj[1][1], wj[1][2]])
            iv = jnp.stack([wj[0][3], wj[0][4], wj[1][3], wj[1][4]])
            msg0, msg1 = _conv_call(EPAD[l], ip, oc, BE_[l])(
                psd[l], gx0, gx1, mu, iv, wj[0][0], wj[1][0], expm, sel)
            if j == 0:
                agg0, agg1, deg = _scatter_call(NPAD[l], oc, EPAD[l], True)(
                    msg0, msg1, dst2d[l])
                bnxt = jnp.stack([W[1][0][6], W[1][1][6]])
                x0, x1, rt0, rt1 = _fin_call(NPAD[l], oc, oc)(
                    agg0, agg1, deg, rts[0], rts[1],
                    W[1][0][5], W[1][1][5], bnxt)
                tbls = [x0, x1]
                rts = (rt0, rt1)
            else:
                agg0, agg1 = _scatter_call(NPAD[l], oc, EPAD[l], False)(
                    msg0, msg1, dst2d[l])
                x0, x1 = _fin_call(NPAD[l], oc, None)(
                    agg0, agg1, deg, rts[0], rts[1])
                tbls = [x0, x1]
        if l < 4:
            npl = _ru(NS_[l + 1], 1024)
            xp0, xp1 = _pool_call(NPAD[l], oc, npl)(tbls[0], tbls[1], hexidx[l])
            dnew = NS_[l + 1]
            in_next = 2 * oc + 2
            ipn = _ru(in_next, 16)
            oc2 = NF_[l + 1]
            nm2 = (X_NAMES[2 * l + 2], Y_NAMES[2 * l + 2])
            Wn = [_prep(params, nm2[b], ipn, oc2) for b in range(2)]
            newt = []
            for b in range(2):
                t = jnp.concatenate(
                    [tbls[b][:dnew, :oc], (xp0, xp1)[b][:dnew],
                     inp_b[b][:dnew]], axis=1)
                t = jnp.pad(t, ((0, NPAD[l + 1] - dnew), (0, ipn - in_next)))
                newt.append(t)
            tbls = newt
            rts = _root_call(NPAD[l + 1], ipn, oc2)(
                tbls[0], tbls[1], Wn[0][5], Wn[1][5],
                jnp.stack([Wn[0][6], Wn[1][6]]))
    return tbls[0][:NS_[4]], tbls[1][:NS_[4]]


# speedup vs baseline: 1.5427x; 1.0026x over previous
import functools

import jax
import jax.numpy as jnp
import numpy as np
from jax import lax
from jax.experimental import pallas as pl
from jax.experimental.pallas import tpu as pltpu
from jax.experimental.pallas import tpu_sc as plsc

F32 = jnp.float32
I32 = jnp.int32

NS_ = [40962, 10242, 2562, 642, 162]
ES_ = [6 * n for n in NS_]
NK = 10
KP = 16
NF_ = [16, 32, 64, 128, 256]
EPS = 1e-15
SPMEM_WORDS = 2097151


def _ru(x, m):
    return (x + m - 1) // m * m


EPAD = [_ru(e, 4096) for e in ES_]
NPAD = [_ru(n + 8, 2048) for n in NS_]
BE_ = [4096, 4096, 2048, 1024, 512]

X_NAMES = ['conv1', 'conv1s', 'conv2', 'conv2s', 'conv3', 'conv3s',
           'conv4', 'conv4s', 'conv5', 'conv5s']
Y_NAMES = ['conv1_d', 'conv1s_d', 'conv2_d', 'conv2s_d', 'conv3_d',
           'conv3s_d', 'conv4', 'conv4s', 'conv5', 'conv5s']

_MESH = dict(core_axis_name="c", subcore_axis_name="s")


def _nbuf(w):
    return max(1, min(8, 376832 // (512 * w)))


@functools.cache
def _gather_call(npad, w, epad):
    nr_chunks = epad // 128
    ct = nr_chunks // 16
    rows_per_tile = npad // 16
    nb = _nbuf(w)
    ng = ct // nb
    tail = ct - ng * nb

    @functools.partial(
        pl.kernel,
        out_type=(jax.ShapeDtypeStruct((epad, w), F32),
                  jax.ShapeDtypeStruct((epad, w), F32)),
        scratch_types=[
            pltpu.VMEM_SHARED((npad, w), F32),
            pltpu.VMEM((nb, 128), I32),
            pltpu.VMEM((nb * 128, w), F32),
            pltpu.SemaphoreType.DMA,
        ],
        mesh=plsc.VectorSubcoreMesh(**_MESH),
        compiler_params=pltpu.CompilerParams(use_tc_tiling_on_sc=False),
    )
    def gather_k(x0, x1, src2d, gx0, gx1, sh_x, idx2, rows_v, sem):
        cid = lax.axis_index("c")
        sid = lax.axis_index("s")
        base = sid * rows_per_tile

        @pl.when(cid == 0)
        def _():
            pltpu.sync_copy(x0.at[pl.ds(base, rows_per_tile)],
                            sh_x.at[pl.ds(base, rows_per_tile)])

        @pl.when(cid == 1)
        def _():
            pltpu.sync_copy(x1.at[pl.ds(base, rows_per_tile)],
                            sh_x.at[pl.ds(base, rows_per_tile)])
        plsc.subcore_barrier()

        def group(gx, r0, k):
            pltpu.sync_copy(src2d.at[pl.ds(r0, k)], idx2.at[pl.ds(0, k)])
            descs = [pltpu.async_copy(sh_x.at[idx2.at[jb]],
                                      rows_v.at[pl.ds(jb * 128, 128)], sem)
                     for jb in range(k)]
            for d in descs:
                d.wait()
            pltpu.sync_copy(rows_v.at[pl.ds(0, k * 128)],
                            gx.at[pl.ds(r0 * 128, k * 128)])

        def pipe(gx):
            def body(gi, _):
                group(gx, sid * ct + gi * nb, nb)
                return 0
            lax.fori_loop(0, ng, body, 0)
            if tail:
                group(gx, sid * ct + ng * nb, tail)

        @pl.when(cid == 0)
        def _():
            pipe(gx0)

        @pl.when(cid == 1)
        def _():
            pipe(gx1)

    return gather_k


@functools.cache
def _scatter_call(npad, oc, epad, first):
    nr_chunks = epad // 128
    ct = nr_chunks // 16
    rows_per_tile = npad // 16
    zch = rows_per_tile // 128

    sh_words = npad * oc + (npad * 16 if first else 0)
    per_tile_fixed = 128 * oc + 8 * 128 + (2 * 128 * 16 if first else 0) + 1024
    avail = SPMEM_WORDS - sh_words - 16 * per_tile_fixed - 16384
    nb = max(1, min(8, avail // (16 * 128 * oc)))
    ng = ct // nb
    tail = ct - ng * nb

    out_type = [jax.ShapeDtypeStruct((npad, oc), F32),
                jax.ShapeDtypeStruct((npad, oc), F32)]
    scratch = [
        pltpu.VMEM_SHARED((npad, oc), F32),
        pltpu.VMEM((128, oc), F32),
        pltpu.VMEM((nb, 128), I32),
        pltpu.VMEM((nb * 128, oc), F32),
        pltpu.SemaphoreType.DMA,
        pltpu.SemaphoreType.DMA,
    ]
    if first:
        out_type.append(jax.ShapeDtypeStruct((npad, 16), F32))
        scratch.append(pltpu.VMEM_SHARED((npad, 16), F32))
        scratch.append(pltpu.VMEM((128, 16), F32))
        scratch.append(pltpu.VMEM((128, 16), F32))

    @functools.partial(
        pl.kernel,
        out_type=tuple(out_type),
        scratch_types=scratch,
        mesh=plsc.VectorSubcoreMesh(**_MESH),
        compiler_params=pltpu.CompilerParams(use_tc_tiling_on_sc=False),
    )
    def scatter_k(msg0, msg1, dst2d, agg0, agg1, *rest):
        if first:
            dego, sh_agg, zbuf, idx2, rows_v, sem, sem2, sh_deg, ones_v, zbuf16 = rest
        else:
            sh_agg, zbuf, idx2, rows_v, sem, sem2 = rest
        cid = lax.axis_index("c")
        sid = lax.axis_index("s")
        base = sid * rows_per_tile

        def zrow(r, _):
            for c in range(oc // 16):
                zbuf[r, pl.ds(c * 16, 16)] = jnp.zeros((16,), F32)
            if first:
                ones_v[r] = jnp.ones((16,), F32)
                zbuf16[r] = jnp.zeros((16,), F32)
            return 0

        lax.fori_loop(0, 128, zrow, 0)

        descs = [pltpu.async_copy(zbuf, sh_agg.at[pl.ds(base + q * 128, 128)],
                                  sem)
                 for q in range(zch)]
        if first:
            @pl.when(cid == 0)
            def _():
                dd = [pltpu.async_copy(
                    zbuf16, sh_deg.at[pl.ds(base + q * 128, 128)], sem2)
                    for q in range(zch)]
                for d in dd:
                    d.wait()
        for d in descs:
            d.wait()
        plsc.subcore_barrier()

        def group(msg, r0, k, deg_too):
            din = [pltpu.async_copy(dst2d.at[pl.ds(r0, k)],
                                    idx2.at[pl.ds(0, k)], sem),
                   pltpu.async_copy(msg.at[pl.ds(r0 * 128, k * 128)],
                                    rows_v.at[pl.ds(0, k * 128)], sem)]
            for d in din:
                d.wait()
            descs = [pltpu.async_copy(rows_v.at[pl.ds(jb * 128, 128)],
                                      sh_agg.at[idx2.at[jb]], sem, add=True)
                     for jb in range(k)]
            if deg_too:
                dd = [pltpu.async_copy(ones_v, sh_deg.at[idx2.at[jb]],
                                       sem2, add=True)
                      for jb in range(k)]
                for d in dd:
                    d.wait()
            for d in descs:
                d.wait()

        def pipe(msg, deg_too):
            def body(gi, _):
                group(msg, sid * ct + gi * nb, nb, deg_too)
                return 0
            lax.fori_loop(0, ng, body, 0)
            if tail:
                group(msg, sid * ct + ng * nb, tail, deg_too)

        @pl.when(cid == 0)
        def _():
            pipe(msg0, first)

        @pl.when(cid == 1)
        def _():
            pipe(msg1, False)

        plsc.subcore_barrier()

        @pl.when(cid == 0)
        def _():
            dd = [pltpu.async_copy(sh_agg.at[pl.ds(base + q * 128, 128)],
                                   agg0.at[pl.ds(base + q * 128, 128)], sem)
                  for q in range(zch)]
            if first:
                dd += [pltpu.async_copy(sh_deg.at[pl.ds(base + q * 128, 128)],
                                        dego.at[pl.ds(base + q * 128, 128)],
                                        sem2)
                       for q in range(zch)]
            for d in dd:
                d.wait()

        @pl.when(cid == 1)
        def _():
            dd = [pltpu.async_copy(sh_agg.at[pl.ds(base + q * 128, 128)],
                                   agg1.at[pl.ds(base + q * 128, 128)], sem)
                  for q in range(zch)]
            for d in dd:
                d.wait()

    return scatter_k


@functools.cache
def _pool_call(npad_prev, f, npool):
    ch = npool // 64
    cpt = ch // 16
    lf = int(np.log2(f))
    rows_per_tile = npad_prev // 16
    staged = (SPMEM_WORDS - npad_prev * f
              - 16 * (512 * f + 64 * f + 1024) - 16384) >= 0

    scratch = [
        pltpu.VMEM((4, 128), I32),
        pltpu.VMEM((512, f), F32),
        pltpu.VMEM((64, f), F32),
        pltpu.SemaphoreType.DMA,
    ]
    if staged:
        scratch.append(pltpu.VMEM_SHARED((npad_prev, f), F32))

    @functools.partial(
        pl.kernel,
        out_type=(jax.ShapeDtypeStruct((npool, f), F32),
                  jax.ShapeDtypeStruct((npool, f), F32)),
        scratch_types=scratch,
        mesh=plsc.VectorSubcoreMesh(**_MESH),
        compiler_params=pltpu.CompilerParams(use_tc_tiling_on_sc=False,
                                             needs_layout_passes=False),
    )
    def pool_k(x0, x1, hexidx, xp0, xp1, idx_v, rows_v, out_v, sem, *sh):
        cid = lax.axis_index("c")
        sid = lax.axis_index("s")
        iot7 = 7 * lax.iota(I32, 16)
        if staged:
            sh_x = sh[0]
            base = sid * rows_per_tile

            @pl.when(cid == 0)
            def _():
                pltpu.sync_copy(x0.at[pl.ds(base, rows_per_tile)],
                                sh_x.at[pl.ds(base, rows_per_tile)])

            @pl.when(cid == 1)
            def _():
                pltpu.sync_copy(x1.at[pl.ds(base, rows_per_tile)],
                                sh_x.at[pl.ds(base, rows_per_tile)])
            plsc.subcore_barrier()

        def body(j, _):
            chn = sid * cpt + j
            pltpu.sync_copy(hexidx.at[pl.ds(chn * 4, 4)], idx_v)

            def fetch(src):
                dd = [pltpu.async_copy(
                    src.at[idx_v.at[q]], rows_v.at[pl.ds(q * 128, 128)], sem)
                    for q in range(4)]
                for d in dd:
                    d.wait()

            if staged:
                fetch(sh_x)
            else:
                @pl.when(cid == 0)
                def _():
                    fetch(x0)

                @pl.when(cid == 1)
                def _():
                    fetch(x1)

            def node(i, _):
                for c in range(f // 16):
                    acc = None
                    for jj in range(7):
                        p = 112 * c + jj + iot7
                        row = 7 * i + (p >> lf)
                        col = p & (f - 1)
                        v = plsc.load_gather(rows_v, [row, col])
                        acc = v if acc is None else jnp.maximum(acc, v)
                    out_v[i, pl.ds(c * 16, 16)] = acc
                return 0

            lax.fori_loop(0, 64, node, 0)

            @pl.when(cid == 0)
            def _():
                pltpu.sync_copy(out_v, xp0.at[pl.ds(chn * 64, 64)])

            @pl.when(cid == 1)
            def _():
                pltpu.sync_copy(out_v, xp1.at[pl.ds(chn * 64, 64)])
            return 0

        lax.fori_loop(0, cpt, body, 0)

    return pool_k


@functools.cache
def _conv_call(epad, inpad, oc, be):
    ko = KP * oc

    def body(psd, gx0, gx1, mu, iv, g0, g1, expm, sel, msg0, msg1):
        p = psd[...]
        p0 = p[:, 0:1]
        p1 = p[:, 1:2]
        mua = mu[...]
        iva = iv[...]
        ea = expm[...]
        sa = sel[...]
        for b in range(2):
            gx = (gx0, gx1)[b][...]
            g = (g0, g1)[b][...]
            m0 = mua[2 * b:2 * b + 1, :]
            m1 = mua[2 * b + 1:2 * b + 2, :]
            i0 = iva[2 * b:2 * b + 1, :]
            i1 = iva[2 * b + 1:2 * b + 2, :]
            w = jnp.exp(-0.5 * ((p0 - m0) ** 2 * i0 + (p1 - m1) ** 2 * i1))
            wexp = jnp.dot(w, ea, preferred_element_type=F32)
            xj = jnp.dot(gx, g, preferred_element_type=F32)
            (msg0, msg1)[b][...] = jnp.dot(xj * wexp, sa,
                                           preferred_element_type=F32)

    const = lambda i: (0, 0)
    row = lambda i: (i, 0)
    return pl.pallas_call(
        body,
        grid=(epad // be,),
        in_specs=[
            pl.BlockSpec((be, 2), row),
            pl.BlockSpec((be, inpad), row),
            pl.BlockSpec((be, inpad), row),
            pl.BlockSpec((4, KP), const),
            pl.BlockSpec((4, KP), const),
            pl.BlockSpec((inpad, ko), const),
            pl.BlockSpec((inpad, ko), const),
            pl.BlockSpec((KP, ko), const),
            pl.BlockSpec((ko, oc), const),
        ],
        out_specs=[pl.BlockSpec((be, oc), row)] * 2,
        out_shape=[jax.ShapeDtypeStruct((epad, oc), F32)] * 2,
    )


@functools.cache
def _root_call(npad, w, oc, bn=1024):
    def body(t0, t1, r0, r1, bias, rt0, rt1):
        ba = bias[...]
        for b in range(2):
            t = (t0, t1)[b][...]
            r = (r0, r1)[b][...]
            (rt0, rt1)[b][...] = (jnp.dot(t, r, preferred_element_type=F32)
                                  + ba[b:b + 1, :])

    const = lambda i: (0, 0)
    row = lambda i: (i, 0)
    return pl.pallas_call(
        body,
        grid=(npad // bn,),
        in_specs=[
            pl.BlockSpec((bn, w), row),
            pl.BlockSpec((bn, w), row),
            pl.BlockSpec((w, oc), const),
            pl.BlockSpec((w, oc), const),
            pl.BlockSpec((2, oc), const),
        ],
        out_specs=[pl.BlockSpec((bn, oc), row)] * 2,
        out_shape=[jax.ShapeDtypeStruct((npad, oc), F32)] * 2,
    )


@functools.cache
def _fin_call(npad, oc, ocn, bn=1024):
    mid = ocn is not None

    def body(a0, a1, deg, rt0, rt1, *rest):
        if mid:
            rn0, rn1, bnxt, x0, x1, xr0, xr1 = rest
        else:
            x0, x1 = rest
        d = jnp.maximum(deg[...][:, 0:1], 1.0)
        for b in range(2):
            x = (a0, a1)[b][...] / d + (rt0, rt1)[b][...]
            x = jnp.maximum(x, 0.2 * x)
            (x0, x1)[b][...] = x
            if mid:
                (xr0, xr1)[b][...] = (
                    jnp.dot(x, (rn0, rn1)[b][...], preferred_element_type=F32)
                    + bnxt[...][b:b + 1, :])

    const = lambda i: (0, 0)
    row = lambda i: (i, 0)
    in_specs = [
        pl.BlockSpec((bn, oc), row),
        pl.BlockSpec((bn, oc), row),
        pl.BlockSpec((bn, 16), row),
        pl.BlockSpec((bn, oc), row),
        pl.BlockSpec((bn, oc), row),
    ]
    out_specs = [pl.BlockSpec((bn, oc), row)] * 2
    out_shape = [jax.ShapeDtypeStruct((npad, oc), F32)] * 2
    if mid:
        in_specs += ([pl.BlockSpec((oc, ocn), const)] * 2
                     + [pl.BlockSpec((2, ocn), const)])
        out_specs += [pl.BlockSpec((bn, ocn), row)] * 2
        out_shape += [jax.ShapeDtypeStruct((npad, ocn), F32)] * 2
    return pl.pallas_call(
        body,
        grid=(npad // bn,),
        in_specs=in_specs,
        out_specs=out_specs,
        out_shape=out_shape,
    )


@functools.cache
def _expand_sel(oc):
    e = np.zeros((KP, KP * oc), np.float32)
    s = np.zeros((KP * oc, oc), np.float32)
    for k in range(KP):
        e[k, k * oc:(k + 1) * oc] = 1.0
        s[k * oc:(k + 1) * oc, :] = np.eye(oc, dtype=np.float32)
    return jnp.asarray(e), jnp.asarray(s)


def _prep(params, name, inpad, oc):
    p = params[name]
    ic = p['g'].shape[0]
    g = jnp.zeros((inpad, KP * oc), F32).at[:ic, :NK * oc].set(p['g'])
    iv = 1.0 / (p['sigma'] ** 2 + EPS)
    mu0 = jnp.zeros((KP,), F32).at[:NK].set(p['mu'][:, 0])
    mu1 = jnp.zeros((KP,), F32).at[:NK].set(p['mu'][:, 1])
    iv0 = jnp.zeros((KP,), F32).at[:NK].set(iv[:, 0])
    iv1 = jnp.zeros((KP,), F32).at[:NK].set(iv[:, 1])
    root = jnp.zeros((inpad, oc), F32).at[:ic].set(p['root'])
    return g, mu0, mu1, iv0, iv1, root, p['bias']


def kernel(moving, target, edge_input, params,
           edge_index1, edge_index2, edge_index3, edge_index4,
           pseudo0, pseudo1, pseudo2, pseudo3, pseudo4,
           hex0, hex1, hex2, hex3):
    edges = [edge_input, edge_index1, edge_index2, edge_index3, edge_index4]
    pseudos = [pseudo0, pseudo1, pseudo2, pseudo3, pseudo4]
    hexes = [hex0, hex1, hex2, hex3]
    inp_b = [moving, target]

    src2d, dst2d, psd = [], [], []
    for l in range(5):
        e, ep = ES_[l], EPAD[l]
        s = jnp.zeros((ep,), I32).at[:e].set(edges[l][0])
        t = jnp.full((ep,), NS_[l], I32).at[:e].set(edges[l][1])
        src2d.append(s.reshape(ep // 128, 128))
        dst2d.append(t.reshape(ep // 128, 128))
        psd.append(jnp.zeros((ep, 2), F32).at[:e].set(pseudos[l]))

    hexidx = []
    for l in range(4):
        npl = _ru(NS_[l + 1], 1024)
        h = jnp.zeros((npl, 7), I32).at[:NS_[l + 1]].set(hexes[l])
        h = jnp.pad(h.reshape(npl // 64, 448), ((0, 0), (0, 64)))
        hexidx.append(h.reshape(npl // 64 * 4, 128))

    tbls = [jnp.zeros((NPAD[0], 16), F32).at[:NS_[0], :2].set(inp_b[b])
            for b in range(2)]
    rts = None

    for l in range(5):
        oc = NF_[l]
        in0 = 2 if l == 0 else 2 * NF_[l - 1] + 2
        inpads = [_ru(in0, 16), oc]
        names = [(X_NAMES[2 * l], Y_NAMES[2 * l]),
                 (X_NAMES[2 * l + 1], Y_NAMES[2 * l + 1])]
        W = [[_prep(params, names[j][b], inpads[j], oc) for b in range(2)]
             for j in range(2)]
        if l == 0:
            rts = _root_call(NPAD[0], 16, oc)(
                tbls[0], tbls[1], W[0][0][5], W[0][1][5],
                jnp.stack([W[0][0][6], W[0][1][6]]))
        expm, sel = _expand_sel(oc)
        deg = None
        for j in (0, 1):
            wj = W[j]
            ip = inpads[j]
            gx0, gx1 = _gather_call(NPAD[l], ip, EPAD[l])(
                tbls[0], tbls[1], src2d[l])
            mu = jnp.stack([wj[0][1], wj[0][2], wj[1][1], wj[1][2]])
            iv = jnp.stack([wj[0][3], wj[0][4], wj[1][3], wj[1][4]])
            msg0, msg1 = _conv_call(EPAD[l], ip, oc, BE_[l])(
                psd[l], gx0, gx1, mu, iv, wj[0][0], wj[1][0], expm, sel)
            if j == 0:
                agg0, agg1, deg = _scatter_call(NPAD[l], oc, EPAD[l], True)(
                    msg0, msg1, dst2d[l])
                bnxt = jnp.stack([W[1][0][6], W[1][1][6]])
                x0, x1, rt0, rt1 = _fin_call(NPAD[l], oc, oc)(
                    agg0, agg1, deg, rts[0], rts[1],
                    W[1][0][5], W[1][1][5], bnxt)
                tbls = [x0, x1]
                rts = (rt0, rt1)
            else:
                agg0, agg1 = _scatter_call(NPAD[l], oc, EPAD[l], False)(
                    msg0, msg1, dst2d[l])
                x0, x1 = _fin_call(NPAD[l], oc, None)(
                    agg0, agg1, deg, rts[0], rts[1])
                tbls = [x0, x1]
        if l < 4:
            npl = _ru(NS_[l + 1], 1024)
            xp0, xp1 = _pool_call(NPAD[l], oc, npl)(tbls[0], tbls[1], hexidx[l])
            dnew = NS_[l + 1]
            in_next = 2 * oc + 2
            ipn = _ru(in_next, 16)
            oc2 = NF_[l + 1]
            nm2 = (X_NAMES[2 * l + 2], Y_NAMES[2 * l + 2])
            Wn = [_prep(params, nm2[b], ipn, oc2) for b in range(2)]
            newt = []
            for b in range(2):
                t = jnp.concatenate(
                    [tbls[b][:dnew, :oc], (xp0, xp1)[b][:dnew],
                     inp_b[b][:dnew]], axis=1)
                t = jnp.pad(t, ((0, NPAD[l + 1] - dnew), (0, ipn - in_next)))
                newt.append(t)
            tbls = newt
            rts = _root_call(NPAD[l + 1], ipn, oc2)(
                tbls[0], tbls[1], Wn[0][5], Wn[1][5],
                jnp.stack([Wn[0][6], Wn[1][6]]))
    return tbls[0][:NS_[4]], tbls[1][:NS_[4]]
